# baseline (speedup 1.0000x reference)
import jax
import jax.numpy as jnp
from jax.experimental import pallas as pl
from jax.experimental.pallas import tpu as pltpu

_BM = 2048


def _split_cat0(x):
    hi = x.astype(jnp.bfloat16)
    lo = (x - hi.astype(jnp.float32)).astype(jnp.bfloat16)
    return jnp.concatenate([hi, lo], axis=0)


def _layer_t(xT, wt_ref, b_ref):
    n = wt_ref.shape[0] // 2
    r = jnp.dot(wt_ref[...], _split_cat0(xT), preferred_element_type=jnp.float32)
    return jnp.tanh(r[:n] + r[n:] + b_ref[...])


def _make_body(bm, steps):
    def body(obs_ref, w1t_ref, w2t_ref, b1_ref, b2_ref, out_ref, buf, sem):
        i = pl.program_id(0)

        def copy(tile, slot):
            return pltpu.make_async_copy(
                obs_ref.at[pl.ds(tile * bm, bm), :], buf.at[slot], sem.at[slot]
            )

        @pl.when(i == 0)
        def _prologue():
            copy(0, 0).start()

        @pl.when(i + 1 < steps)
        def _prefetch():
            copy(i + 1, (i + 1) % 2).start()

        copy(i, i % 2).wait()
        obsT = buf[i % 2].T
        hT = _layer_t(obsT, w1t_ref, b1_ref)
        out_ref[...] = _layer_t(hT, w2t_ref, b2_ref)

    return body


def _pack_weights_t(w):
    hi = w.astype(jnp.bfloat16)
    lo = (w - hi.astype(jnp.float32)).astype(jnp.bfloat16)
    half = jnp.concatenate([hi, lo], axis=1)
    packed = jnp.concatenate([half, half], axis=0)
    return packed.T


def kernel(obs, W1, W2, b1, b2):
    if obs.ndim == 1:
        obs = obs[None, :]
    batch, n_in = obs.shape
    n_hid = W1.shape[1]
    n_out = W2.shape[1]
    w1t = _pack_weights_t(W1)
    w2t = _pack_weights_t(W2)
    bm = min(_BM, batch)
    steps = pl.cdiv(batch, bm)
    rep = lambda i: (0, 0)
    out = pl.pallas_call(
        _make_body(bm, steps),
        grid=(steps,),
        in_specs=[
            pl.BlockSpec(memory_space=pltpu.MemorySpace.HBM),
            pl.BlockSpec((2 * n_hid, 2 * n_in), rep),
            pl.BlockSpec((2 * n_out, 2 * n_hid), rep),
            pl.BlockSpec((n_hid, 1), rep),
            pl.BlockSpec((n_out, 1), rep),
        ],
        out_specs=pl.BlockSpec((n_out, bm), lambda i: (0, i)),
        out_shape=jax.ShapeDtypeStruct((n_out, batch), jnp.float32),
        scratch_shapes=[
            pltpu.VMEM((2, bm, n_in), jnp.float32),
            pltpu.SemaphoreType.DMA((2,)),
        ],
    )(obs, w1t, w2t, b1[:, None], b2[:, None])
    return out.T


# speedup vs baseline: 1.1533x; 1.1533x over previous
import jax
import jax.numpy as jnp
from jax.experimental import pallas as pl
from jax.experimental.pallas import tpu as pltpu

_BM = 4096


def _split_cat0(x):
    hi = x.astype(jnp.bfloat16)
    lo = (x - hi.astype(jnp.float32)).astype(jnp.bfloat16)
    return jnp.concatenate([hi, lo], axis=0)


def _layer_t(xT, wt_ref, b_ref):
    n = wt_ref.shape[0] // 2
    r = jnp.dot(wt_ref[...], _split_cat0(xT), preferred_element_type=jnp.float32)
    return jnp.tanh(r[:n] + r[n:] + b_ref[...])


def _make_body(bm, steps):
    def body(obs_ref, w1t_ref, w2t_ref, b1_ref, b2_ref, out_ref, buf, sem):
        i = pl.program_id(0)

        def copy(tile, slot):
            return pltpu.make_async_copy(
                obs_ref.at[pl.ds(tile * bm, bm), :], buf.at[slot], sem.at[slot]
            )

        @pl.when(i == 0)
        def _prologue():
            copy(0, 0).start()

        @pl.when(i + 1 < steps)
        def _prefetch():
            copy(i + 1, (i + 1) % 2).start()

        copy(i, i % 2).wait()
        obsT = buf[i % 2].T
        hT = _layer_t(obsT, w1t_ref, b1_ref)
        out_ref[...] = _layer_t(hT, w2t_ref, b2_ref)

    return body


def _pack_weights_t(w):
    hi = w.astype(jnp.bfloat16)
    lo = (w - hi.astype(jnp.float32)).astype(jnp.bfloat16)
    half = jnp.concatenate([hi, lo], axis=1)
    packed = jnp.concatenate([half, half], axis=0)
    return packed.T


def kernel(obs, W1, W2, b1, b2):
    if obs.ndim == 1:
        obs = obs[None, :]
    batch, n_in = obs.shape
    n_hid = W1.shape[1]
    n_out = W2.shape[1]
    w1t = _pack_weights_t(W1)
    w2t = _pack_weights_t(W2)
    bm = min(_BM, batch)
    steps = pl.cdiv(batch, bm)
    rep = lambda i: (0, 0)
    out = pl.pallas_call(
        _make_body(bm, steps),
        grid=(steps,),
        in_specs=[
            pl.BlockSpec(memory_space=pltpu.MemorySpace.HBM),
            pl.BlockSpec((2 * n_hid, 2 * n_in), rep),
            pl.BlockSpec((2 * n_out, 2 * n_hid), rep),
            pl.BlockSpec((n_hid, 1), rep),
            pl.BlockSpec((n_out, 1), rep),
        ],
        out_specs=pl.BlockSpec((n_out, bm), lambda i: (0, i)),
        out_shape=jax.ShapeDtypeStruct((n_out, batch), jnp.float32),
        scratch_shapes=[
            pltpu.VMEM((2, bm, n_in), jnp.float32),
            pltpu.SemaphoreType.DMA((2,)),
        ],
    )(obs, w1t, w2t, b1[:, None], b2[:, None])
    return out.T
